# baseline (speedup 1.0000x reference)
import jax
import jax.numpy as jnp
from jax import lax
from jax.experimental import pallas as pl
from jax.experimental.pallas import tpu as pltpu
from jax.experimental.pallas import tpu_sc as plsc

_NUM_BUCKETS = 32
_H = 16
_Q = 2048
_K = 2048
_LINE = 4224
_NS = 16


def _line_body(scal_ref, table_ref, out_ref):
    h = pl.program_id(0)
    delta = scal_ref[0]
    boff = scal_ref[1]
    u = jax.lax.broadcasted_iota(jnp.int32, (1, _LINE), 1)
    rel = (2047 - u) + delta
    neg16 = jnp.where(rel < 0, 16, 0)
    n = jnp.abs(rel)
    nf = n.astype(jnp.float32)
    val_large = 8 + (jnp.log(nf / 8.0) / jnp.log(16.0) * 8.0).astype(jnp.int32)
    val_large = jnp.minimum(val_large, 15)
    bucket = neg16 + jnp.where(n < 8, n, val_large) + boff
    idx = jnp.mod(bucket, _NUM_BUCKETS)
    line = jnp.zeros((1, _LINE), jnp.float32)
    for b in range(_NUM_BUCKETS):
        line = jnp.where(idx == b, table_ref[h, b], line)
    for t in range(_NS):
        row = line if t == 0 else jnp.concatenate(
            [jnp.zeros((1, t), jnp.float32), line[:, : _LINE - t]], axis=1)
        out_ref[0, pl.ds(t, 1), :] = row


def _sc_expand_body(w16_hbm, out_hbm, stk, sem):
    c = lax.axis_index("c")
    s = lax.axis_index("s")
    wid = s * 2 + c
    h = wid // 2
    rbase = (wid % 2) * (_Q // 2)
    pltpu.sync_copy(w16_hbm.at[h], stk)

    def group(g, carry):
        i0 = rbase + g * 8
        handles = []
        for u in range(8):
            i = i0 + u
            t = lax.rem(i + 1, _NS)
            o16 = (2047 - i) + t
            src_off = pl.multiple_of(t * _LINE + o16, 16)
            dst_off = pl.multiple_of((h * _Q + i) * _K, 2048)
            handles.append(pltpu.async_copy(
                stk.at[pl.ds(src_off, _K)],
                out_hbm.at[pl.ds(dst_off, _K)], sem))
        for hd in handles:
            hd.wait()
        return carry

    lax.fori_loop(0, (_Q // 2) // 8, group, 0)


def kernel(q_len, k_len, bidirectional, relative_attention_bias):
    delta = jnp.asarray(q_len, jnp.int32) - jnp.asarray(k_len, jnp.int32)
    boff = jnp.asarray(bidirectional, jnp.int32) - 1
    scal = jnp.stack([delta, boff])
    w16 = pl.pallas_call(
        _line_body,
        grid=(_H,),
        in_specs=[
            pl.BlockSpec(memory_space=pltpu.SMEM),
            pl.BlockSpec(memory_space=pltpu.SMEM),
        ],
        out_specs=pl.BlockSpec((1, _NS, _LINE), lambda h: (h, 0, 0)),
        out_shape=jax.ShapeDtypeStruct((_H, _NS, _LINE), jnp.float32),
    )(scal, relative_attention_bias)
    w16 = w16.reshape(_H, _NS * _LINE)

    mesh = plsc.VectorSubcoreMesh(core_axis_name="c", subcore_axis_name="s")
    expand = pl.kernel(
        _sc_expand_body,
        out_type=jax.ShapeDtypeStruct((_H * _Q * _K,), jnp.float32),
        mesh=mesh,
        scratch_types=[
            pltpu.VMEM((_NS * _LINE,), jnp.float32),
            pltpu.SemaphoreType.DMA,
        ],
    )
    return expand(w16).reshape(1, _H, _Q, _K)


# speedup vs baseline: 1.0125x; 1.0125x over previous
import jax
import jax.numpy as jnp
from jax import lax
from jax.experimental import pallas as pl
from jax.experimental.pallas import tpu as pltpu
from jax.experimental.pallas import tpu_sc as plsc

_NUM_BUCKETS = 32
_H = 16
_Q = 2048
_K = 2048
_LINE = 4224
_NS = 16


def _line_body(scal_ref, table_ref, out_ref):
    h = pl.program_id(0)
    delta = scal_ref[0]
    boff = scal_ref[1]
    u = jax.lax.broadcasted_iota(jnp.int32, (1, _LINE), 1)
    rel = (2047 - u) + delta
    neg16 = jnp.where(rel < 0, 16, 0)
    n = jnp.abs(rel)
    nf = n.astype(jnp.float32)
    val_large = 8 + (jnp.log(nf / 8.0) / jnp.log(16.0) * 8.0).astype(jnp.int32)
    val_large = jnp.minimum(val_large, 15)
    bucket = neg16 + jnp.where(n < 8, n, val_large) + boff
    idx = jnp.mod(bucket, _NUM_BUCKETS)
    line = jnp.zeros((1, _LINE), jnp.float32)
    for b in range(_NUM_BUCKETS):
        line = jnp.where(idx == b, table_ref[h, b], line)
    for t in range(_NS):
        row = line if t == 0 else jnp.concatenate(
            [jnp.zeros((1, t), jnp.float32), line[:, : _LINE - t]], axis=1)
        out_ref[0, pl.ds(t, 1), :] = row


def _sc_expand_body(w16_hbm, out_hbm, stk, sem):
    c = lax.axis_index("c")
    s = lax.axis_index("s")
    wid = s * 2 + c
    h = wid // 2
    rbase = (wid % 2) * (_Q // 2)
    pltpu.sync_copy(w16_hbm.at[h], stk)

    G = 8
    NG = (_Q // 2) // G

    def fire(g):
        i0 = rbase + g * G
        handles = []
        for u in range(G):
            i = i0 + u
            t = lax.rem(i + 1, _NS)
            o16 = (2047 - i) + t
            src_off = pl.multiple_of(t * _LINE + o16, 16)
            dst_off = pl.multiple_of((h * _Q + i) * _K, 2048)
            handles.append(pltpu.async_copy(
                stk.at[pl.ds(src_off, _K)],
                out_hbm.at[pl.ds(dst_off, _K)], sem))
        return handles

    def body(p, carry):
        ha = fire(2 * p)
        hb = fire(2 * p + 1)
        for hd in ha:
            hd.wait()
        for hd in hb:
            hd.wait()
        return carry

    lax.fori_loop(0, NG // 2, body, 0)


def kernel(q_len, k_len, bidirectional, relative_attention_bias):
    delta = jnp.asarray(q_len, jnp.int32) - jnp.asarray(k_len, jnp.int32)
    boff = jnp.asarray(bidirectional, jnp.int32) - 1
    scal = jnp.stack([delta, boff])
    w16 = pl.pallas_call(
        _line_body,
        grid=(_H,),
        in_specs=[
            pl.BlockSpec(memory_space=pltpu.SMEM),
            pl.BlockSpec(memory_space=pltpu.SMEM),
        ],
        out_specs=pl.BlockSpec((1, _NS, _LINE), lambda h: (h, 0, 0)),
        out_shape=jax.ShapeDtypeStruct((_H, _NS, _LINE), jnp.float32),
    )(scal, relative_attention_bias)
    w16 = w16.reshape(_H, _NS * _LINE)

    mesh = plsc.VectorSubcoreMesh(core_axis_name="c", subcore_axis_name="s")
    expand = pl.kernel(
        _sc_expand_body,
        out_type=jax.ShapeDtypeStruct((_H * _Q * _K,), jnp.float32),
        mesh=mesh,
        scratch_types=[
            pltpu.VMEM((_NS * _LINE,), jnp.float32),
            pltpu.SemaphoreType.DMA,
        ],
    )
    return expand(w16).reshape(1, _H, _Q, _K)
